# baseline (speedup 1.0000x reference)
import jax
import jax.numpy as jnp
import numpy as np
from jax import lax
from jax.experimental import pallas as pl
from jax.experimental.pallas import tpu as pltpu
from jax.experimental.pallas import tpu_sc as plsc

_UNPRUNED = 1_000_000
_N = 16384 * 26
_NW = 32
_NT = _N // _NW
_NCHUNK = 4
_CH = _NT // _NCHUNK
_ZEROS = np.zeros((_UNPRUNED,), np.float32)


def _sc_body(idx_hbm, al_hbm, ru_hbm, zero_hbm, remap_hbm, p0_hbm, p1_hbm,
             idx_v, val_v, ones_v, ru_shared, sem_i, sem_g, sem_s, stage_sem):
  c = lax.axis_index("c")
  s = lax.axis_index("s")
  base = (c * 16 + s) * _NT

  pltpu.async_copy(idx_hbm.at[pl.ds(base, _NT)], idx_v, sem_i)

  @pl.when((s == 0) & (c == 0))
  def _():
    pltpu.async_copy(ru_hbm, ru_shared, stage_sem)

  @pl.when((s == 0) & (c == 1))
  def _():
    pltpu.async_copy(zero_hbm, ru_shared, stage_sem)

  def _fill(i, carry):
    ones_v[pl.ds(i * 16, 16)] = jnp.full((16,), 1.0, jnp.float32)
    return carry
  lax.fori_loop(0, _CH // 16, _fill, 0)

  pltpu.make_async_copy(idx_hbm.at[pl.ds(base, _NT)], idx_v, sem_i).wait()
  gather_cp = pltpu.async_copy(al_hbm.at[idx_v], val_v, sem_g)

  @pl.when(s == 0)
  def _():
    pltpu.make_async_copy(ru_hbm, ru_shared, stage_sem).wait()
  plsc.subcore_barrier()

  scatter_cps = [
      pltpu.async_copy(ones_v, ru_shared.at[idx_v.at[pl.ds(j * _CH, _CH)]],
                       sem_s, add=True)
      for j in range(_NCHUNK)
  ]
  gather_cp.wait()
  pltpu.sync_copy(val_v, remap_hbm.at[pl.ds(base, _NT)])
  for cp in scatter_cps:
    cp.wait()

  plsc.subcore_barrier()

  @pl.when((s == 0) & (c == 0))
  def _():
    pltpu.sync_copy(ru_shared, p0_hbm)

  @pl.when((s == 0) & (c == 1))
  def _():
    pltpu.sync_copy(ru_shared, p1_hbm)


@jax.jit
def _sc_call(flat_idx, address_lookup, row_util, zeros):
  mesh = plsc.VectorSubcoreMesh(core_axis_name="c", subcore_axis_name="s")
  return pl.kernel(
      _sc_body,
      out_type=(
          jax.ShapeDtypeStruct((_N,), jnp.int32),
          jax.ShapeDtypeStruct((_UNPRUNED,), jnp.float32),
          jax.ShapeDtypeStruct((_UNPRUNED,), jnp.float32),
      ),
      mesh=mesh,
      scratch_types=[
          pltpu.VMEM((_NT,), jnp.int32),
          pltpu.VMEM((_NT,), jnp.int32),
          pltpu.VMEM((_CH,), jnp.float32),
          pltpu.VMEM_SHARED((_UNPRUNED,), jnp.float32),
          pltpu.SemaphoreType.DMA,
          pltpu.SemaphoreType.DMA,
          pltpu.SemaphoreType.DMA,
          pltpu.SemaphoreType.DMA,
      ],
  )(flat_idx, address_lookup, row_util, zeros)


def kernel(indices, address_lookup, row_util, cur_iter):
  del cur_iter
  flat = indices.T.reshape(-1)
  remapped_flat, p0, p1 = _sc_call(flat, address_lookup, row_util, _ZEROS)
  remapped = remapped_flat.reshape(26, 16384).T
  return remapped, p0 + p1


# speedup vs baseline: 1.0305x; 1.0305x over previous
import jax
import jax.numpy as jnp
from jax import lax
from jax.experimental import pallas as pl
from jax.experimental.pallas import tpu as pltpu
from jax.experimental.pallas import tpu_sc as plsc

_UNPRUNED = 1_000_000
_N = 16384 * 26
_NW = 32
_NT = _N // _NW
_NCHUNK = 4
_CH = _NT // _NCHUNK


def _sc_body(idx_hbm, al_hbm, ru_hbm, remap_hbm, p0_hbm, p1_hbm,
             idx_v, val_v, ones_v, ru_shared, sem_i, sem_g, sem_s, stage_sem):
  c = lax.axis_index("c")
  s = lax.axis_index("s")
  base = (c * 16 + s) * _NT

  pltpu.async_copy(idx_hbm.at[pl.ds(base, _NT)], idx_v, sem_i)

  @pl.when(s == 0)
  def _():
    pltpu.async_copy(ru_hbm, ru_shared, stage_sem)

  def _fill(i, carry):
    ones_v[pl.ds(i * 16, 16)] = jnp.full((16,), 1.0, jnp.float32)
    return carry
  lax.fori_loop(0, _CH // 16, _fill, 0)

  pltpu.make_async_copy(idx_hbm.at[pl.ds(base, _NT)], idx_v, sem_i).wait()
  gather_cp = pltpu.async_copy(al_hbm.at[idx_v], val_v, sem_g)

  @pl.when(s == 0)
  def _():
    pltpu.make_async_copy(ru_hbm, ru_shared, stage_sem).wait()
  plsc.subcore_barrier()

  scatter_cps = [
      pltpu.async_copy(ones_v, ru_shared.at[idx_v.at[pl.ds(j * _CH, _CH)]],
                       sem_s, add=True)
      for j in range(_NCHUNK)
  ]
  gather_cp.wait()
  pltpu.sync_copy(val_v, remap_hbm.at[pl.ds(base, _NT)])
  for cp in scatter_cps:
    cp.wait()

  plsc.subcore_barrier()

  @pl.when((s == 0) & (c == 0))
  def _():
    pltpu.sync_copy(ru_shared, p0_hbm)

  @pl.when((s == 0) & (c == 1))
  def _():
    pltpu.sync_copy(ru_shared, p1_hbm)


@jax.jit
def _sc_call(flat_idx, address_lookup, row_util):
  mesh = plsc.VectorSubcoreMesh(core_axis_name="c", subcore_axis_name="s")
  return pl.kernel(
      _sc_body,
      out_type=(
          jax.ShapeDtypeStruct((_N,), jnp.int32),
          jax.ShapeDtypeStruct((_UNPRUNED,), jnp.float32),
          jax.ShapeDtypeStruct((_UNPRUNED,), jnp.float32),
      ),
      mesh=mesh,
      scratch_types=[
          pltpu.VMEM((_NT,), jnp.int32),
          pltpu.VMEM((_NT,), jnp.int32),
          pltpu.VMEM((_CH,), jnp.float32),
          pltpu.VMEM_SHARED((_UNPRUNED,), jnp.float32),
          pltpu.SemaphoreType.DMA,
          pltpu.SemaphoreType.DMA,
          pltpu.SemaphoreType.DMA,
          pltpu.SemaphoreType.DMA,
      ],
  )(flat_idx, address_lookup, row_util)


def kernel(indices, address_lookup, row_util, cur_iter):
  del cur_iter
  flat = indices.T.reshape(-1)
  remapped_flat, p0, p1 = _sc_call(flat, address_lookup, row_util)
  remapped = remapped_flat.reshape(26, 16384).T
  return remapped, (p0 - row_util) + p1
